# baseline (speedup 1.0000x reference)
import functools

import jax
import jax.numpy as jnp
from jax import lax
from jax.experimental import pallas as pl
from jax.experimental.pallas import tpu as pltpu
from jax.experimental.pallas import tpu_sc as plsc

_B, _L, _D = 1024, 200, 128
_BL = _B * _L
_N_RING, _N_CHARGE, _N_HYB, _N_CHIR = 3, 4, 9, 5
_NCOMB_PAD = 544

_NC, _NS = 2, 16
_NW = _NC * _NS
_PER_W = _BL // _NW
_CHUNK = 64
_NCHUNKS = _PER_W // _CHUNK
_ROWS_PER_TILE = _NCOMB_PAD // _NS
_NBUF = 2


def _sc_body(ring_hbm, charge_hbm, hyb_hbm, chir_hbm,
             wr_hbm, wc_hbm, wh_hbm, wx_hbm,
             out_hbm,
             idx4_v, cidx_v, wr_v, wc_v, wh_v, wx_v, tmp_v, rows_v, wcomb_sh,
             isem, g0, g1, s0, s1):
    cid = lax.axis_index("c")
    sid = lax.axis_index("s")
    wid = sid * _NC + cid
    tile_base = wid * _PER_W

    idx_cp = [
        pltpu.make_async_copy(src.at[pl.ds(tile_base, _PER_W)],
                              idx4_v.at[i], isem)
        for i, src in enumerate((ring_hbm, charge_hbm, hyb_hbm, chir_hbm))
    ]
    for cp in idx_cp:
        cp.start()

    tabs = [wr_v, wc_v, wh_v, wx_v]
    for dst, s_ in zip(tabs, (wr_hbm, wc_hbm, wh_hbm, wx_hbm)):
        pltpu.sync_copy(s_, dst.at[pl.ds(0, s_.shape[0])])

    def build_row(jl, carry):
        j = sid * _ROWS_PER_TILE + jl
        r = j // (_N_CHARGE * _N_HYB * _N_CHIR)
        c = (j // (_N_HYB * _N_CHIR)) % _N_CHARGE
        h = (j // _N_CHIR) % _N_HYB
        x = j % _N_CHIR
        for k in range(_D // 16):
            v = (wr_v[pl.ds(r * _D + 16 * k, 16)]
                 + wc_v[pl.ds(c * _D + 16 * k, 16)]
                 + wh_v[pl.ds(h * _D + 16 * k, 16)]
                 + wx_v[pl.ds(x * _D + 16 * k, 16)])
            tmp_v[jl, pl.ds(16 * k, 16)] = v
        return carry

    lax.fori_loop(0, _ROWS_PER_TILE, build_row, 0)
    pltpu.sync_copy(
        tmp_v, wcomb_sh.at[pl.ds(sid * _ROWS_PER_TILE, _ROWS_PER_TILE)])

    for cp in idx_cp:
        cp.wait()

    def fuse_chunk(g):
        for k in range(_CHUNK // 16):
            s = pl.ds(g * _CHUNK + k * 16, 16)
            cidx_v[s] = ((idx4_v[0, s] * (_N_CHARGE * _N_HYB * _N_CHIR))
                         + (idx4_v[1, s] * (_N_HYB * _N_CHIR))
                         + (idx4_v[2, s] * _N_CHIR)
                         + idx4_v[3, s])

    fuse_chunk(0)
    fuse_chunk(1)

    plsc.subcore_barrier()

    ssems = [s0, s1]
    gsems = [g0, g1]

    def scat(b, g):
        base = tile_base + g * _CHUNK
        return pltpu.make_async_copy(
            rows_v.at[b], out_hbm.at[pl.ds(base, _CHUNK)], ssems[b])

    def gath(b, g):
        return pltpu.make_async_copy(
            wcomb_sh.at[cidx_v.at[pl.ds(g * _CHUNK, _CHUNK)]],
            rows_v.at[b], gsems[b])

    def step(i, carry):
        for b in range(_NBUF):
            g = i * _NBUF + b

            @pl.when(i > 0)
            def _():
                scat(b, g - 2).wait()
                if b == 0:
                    gath(b, g).start()

            if b == 0:
                @pl.when(i == 0)
                def _():
                    gath(b, g).start()
            else:
                gath(b, g).start()

            @pl.when(g + 2 < _NCHUNKS)
            def _():
                fuse_chunk(g + 2)

            if b == 0:
                @pl.when(i > 0)
                def _():
                    gath(1, g - 1).wait()
                    scat(1, g - 1).start()
            else:
                gath(0, g - 1).wait()
                scat(0, g - 1).start()
        return carry

    lax.fori_loop(0, _NCHUNKS // _NBUF, step, 0)
    gath(1, _NCHUNKS - 1).wait()
    scat(1, _NCHUNKS - 1).start()
    scat(0, 0).wait()
    scat(1, 0).wait()


_sc_kernel = functools.partial(
    pl.kernel,
    out_type=jax.ShapeDtypeStruct((_BL, _D), jnp.float32),
    mesh=plsc.VectorSubcoreMesh(core_axis_name="c", subcore_axis_name="s"),
    scratch_types=[
        pltpu.VMEM((4, _PER_W), jnp.int32),
        pltpu.VMEM((_PER_W,), jnp.int32),
        pltpu.VMEM((16 * _D,), jnp.float32),
        pltpu.VMEM((16 * _D,), jnp.float32),
        pltpu.VMEM((16 * _D,), jnp.float32),
        pltpu.VMEM((16 * _D,), jnp.float32),
        pltpu.VMEM((_ROWS_PER_TILE, _D), jnp.float32),
        pltpu.VMEM((_NBUF, _CHUNK, _D), jnp.float32),
        pltpu.VMEM_SHARED((_NCOMB_PAD, _D), jnp.float32),
        pltpu.SemaphoreType.DMA,
        pltpu.SemaphoreType.DMA,
        pltpu.SemaphoreType.DMA,
        pltpu.SemaphoreType.DMA,
        pltpu.SemaphoreType.DMA,
    ],
)(_sc_body)


@jax.jit
def kernel(prop_atom_in_ring, prop_atom_charge, prop_atom_hybridization,
           prop_atom_chirality, W_in_ring, W_charge, W_hybridization,
           W_chirality):
    r = prop_atom_in_ring.astype(jnp.int32).reshape(_BL)
    c = prop_atom_charge.astype(jnp.int32).reshape(_BL)
    h = prop_atom_hybridization.astype(jnp.int32).reshape(_BL)
    x = prop_atom_chirality.astype(jnp.int32).reshape(_BL)
    out = _sc_kernel(r, c, h, x,
                     W_in_ring.reshape(-1), W_charge.reshape(-1),
                     W_hybridization.reshape(-1), W_chirality.reshape(-1))
    return out.reshape(_B, _L, _D)


# speedup vs baseline: 1.0506x; 1.0506x over previous
import functools

import jax
import jax.numpy as jnp
from jax import lax
from jax.experimental import pallas as pl
from jax.experimental.pallas import tpu as pltpu
from jax.experimental.pallas import tpu_sc as plsc

_B, _L, _D = 1024, 200, 128
_BL = _B * _L
_N_RING, _N_CHARGE, _N_HYB, _N_CHIR = 3, 4, 9, 5
_NCOMB_PAD = 544

_NC, _NS = 2, 16
_NW = _NC * _NS
_PER_W = _BL // _NW
_CHUNK = 128
_NCHUNKS = _PER_W // _CHUNK
_ROWS_PER_TILE = _NCOMB_PAD // _NS
_NBUF = 2


def _sc_body(ring_hbm, charge_hbm, hyb_hbm, chir_hbm,
             wr_hbm, wc_hbm, wh_hbm, wx_hbm,
             out_hbm,
             idx4_v, cidx_v, wr_v, wc_v, wh_v, wx_v, tmp_v, rows_v, wcomb_sh,
             isem, tsem, g0, g1, s0, s1):
    cid = lax.axis_index("c")
    sid = lax.axis_index("s")
    wid = sid * _NC + cid
    tile_base = wid * _PER_W

    idx_cp = [
        pltpu.make_async_copy(src.at[pl.ds(tile_base, _PER_W)],
                              idx4_v.at[i], isem)
        for i, src in enumerate((ring_hbm, charge_hbm, hyb_hbm, chir_hbm))
    ]
    for cp in idx_cp:
        cp.start()

    tab_cp = [
        pltpu.make_async_copy(s_, dst.at[pl.ds(0, s_.shape[0])], tsem)
        for dst, s_ in zip((wr_v, wc_v, wh_v, wx_v),
                           (wr_hbm, wc_hbm, wh_hbm, wx_hbm))
    ]
    for cp in tab_cp:
        cp.start()
    for cp in tab_cp:
        cp.wait()

    def build_row(jl, carry):
        j = sid * _ROWS_PER_TILE + jl
        r = j // (_N_CHARGE * _N_HYB * _N_CHIR)
        c = (j // (_N_HYB * _N_CHIR)) % _N_CHARGE
        h = (j // _N_CHIR) % _N_HYB
        x = j % _N_CHIR
        for k in range(_D // 16):
            v = (wr_v[pl.ds(r * _D + 16 * k, 16)]
                 + wc_v[pl.ds(c * _D + 16 * k, 16)]
                 + wh_v[pl.ds(h * _D + 16 * k, 16)]
                 + wx_v[pl.ds(x * _D + 16 * k, 16)])
            tmp_v[jl, pl.ds(16 * k, 16)] = v
        return carry

    lax.fori_loop(0, _ROWS_PER_TILE, build_row, 0)
    pltpu.sync_copy(
        tmp_v, wcomb_sh.at[pl.ds(sid * _ROWS_PER_TILE, _ROWS_PER_TILE)])

    for cp in idx_cp:
        cp.wait()

    def fuse_chunk(g):
        for k in range(_CHUNK // 16):
            s = pl.ds(g * _CHUNK + k * 16, 16)
            cidx_v[s] = ((idx4_v[0, s] * (_N_CHARGE * _N_HYB * _N_CHIR))
                         + (idx4_v[1, s] * (_N_HYB * _N_CHIR))
                         + (idx4_v[2, s] * _N_CHIR)
                         + idx4_v[3, s])

    fuse_chunk(0)
    fuse_chunk(1)

    plsc.subcore_barrier()

    ssems = [s0, s1]
    gsems = [g0, g1]

    def scat(b, g):
        base = tile_base + g * _CHUNK
        return pltpu.make_async_copy(
            rows_v.at[b], out_hbm.at[pl.ds(base, _CHUNK)], ssems[b])

    def gath(b, g):
        return pltpu.make_async_copy(
            wcomb_sh.at[cidx_v.at[pl.ds(g * _CHUNK, _CHUNK)]],
            rows_v.at[b], gsems[b])

    def step(i, carry):
        for b in range(_NBUF):
            g = i * _NBUF + b

            @pl.when(i > 0)
            def _():
                scat(b, g - 2).wait()
                if b == 0:
                    gath(b, g).start()

            if b == 0:
                @pl.when(i == 0)
                def _():
                    gath(b, g).start()
            else:
                gath(b, g).start()

            @pl.when(g + 2 < _NCHUNKS)
            def _():
                fuse_chunk(g + 2)

            if b == 0:
                @pl.when(i > 0)
                def _():
                    gath(1, g - 1).wait()
                    scat(1, g - 1).start()
            else:
                gath(0, g - 1).wait()
                scat(0, g - 1).start()
        return carry

    lax.fori_loop(0, _NCHUNKS // _NBUF, step, 0)
    gath(1, _NCHUNKS - 1).wait()
    scat(1, _NCHUNKS - 1).start()
    scat(0, 0).wait()
    scat(1, 0).wait()


_sc_kernel = functools.partial(
    pl.kernel,
    out_type=jax.ShapeDtypeStruct((_BL, _D), jnp.float32),
    mesh=plsc.VectorSubcoreMesh(core_axis_name="c", subcore_axis_name="s"),
    scratch_types=[
        pltpu.VMEM((4, _PER_W), jnp.int32),
        pltpu.VMEM((_PER_W,), jnp.int32),
        pltpu.VMEM((16 * _D,), jnp.float32),
        pltpu.VMEM((16 * _D,), jnp.float32),
        pltpu.VMEM((16 * _D,), jnp.float32),
        pltpu.VMEM((16 * _D,), jnp.float32),
        pltpu.VMEM((_ROWS_PER_TILE, _D), jnp.float32),
        pltpu.VMEM((_NBUF, _CHUNK, _D), jnp.float32),
        pltpu.VMEM_SHARED((_NCOMB_PAD, _D), jnp.float32),
        pltpu.SemaphoreType.DMA,
        pltpu.SemaphoreType.DMA,
        pltpu.SemaphoreType.DMA,
        pltpu.SemaphoreType.DMA,
        pltpu.SemaphoreType.DMA,
        pltpu.SemaphoreType.DMA,
    ],
)(_sc_body)


@jax.jit
def kernel(prop_atom_in_ring, prop_atom_charge, prop_atom_hybridization,
           prop_atom_chirality, W_in_ring, W_charge, W_hybridization,
           W_chirality):
    r = prop_atom_in_ring.astype(jnp.int32).reshape(_BL)
    c = prop_atom_charge.astype(jnp.int32).reshape(_BL)
    h = prop_atom_hybridization.astype(jnp.int32).reshape(_BL)
    x = prop_atom_chirality.astype(jnp.int32).reshape(_BL)
    out = _sc_kernel(r, c, h, x,
                     W_in_ring.reshape(-1), W_charge.reshape(-1),
                     W_hybridization.reshape(-1), W_chirality.reshape(-1))
    return out.reshape(_B, _L, _D)


# speedup vs baseline: 1.1006x; 1.0476x over previous
import functools

import jax
import jax.numpy as jnp
from jax import lax
from jax.experimental import pallas as pl
from jax.experimental.pallas import tpu as pltpu
from jax.experimental.pallas import tpu_sc as plsc

_B, _L, _D = 1024, 200, 128
_BL = _B * _L
_N_RING, _N_CHARGE, _N_HYB, _N_CHIR = 3, 4, 9, 5
_NCOMB_PAD = 544

_NC, _NS = 2, 16
_NW = _NC * _NS
_PER_W = _BL // _NW
_CHUNK = 128
_NCHUNKS = _PER_W // _CHUNK
_ROWS_PER_TILE = _NCOMB_PAD // _NS
_BROWS = _B // _NW
_NBUF = 2


def _sc_body(ring_hbm, charge_hbm, hyb_hbm, chir_hbm,
             wr_hbm, wc_hbm, wh_hbm, wx_hbm,
             out_hbm,
             idx4_v, cidx_v, wr_v, wc_v, wh_v, wx_v, tmp_v, rows_v, wcomb_sh,
             isem, tsem, g0, g1, s0, s1):
    cid = lax.axis_index("c")
    sid = lax.axis_index("s")
    wid = sid * _NC + cid
    tile_base = wid * _PER_W

    row_base = wid * _BROWS
    idx_cp = [
        pltpu.make_async_copy(src.at[pl.ds(row_base, _BROWS)],
                              idx4_v.at[i], isem)
        for i, src in enumerate((ring_hbm, charge_hbm, hyb_hbm, chir_hbm))
    ]
    for cp in idx_cp:
        cp.start()

    tab_cp = [
        pltpu.make_async_copy(s_, dst.at[pl.ds(0, s_.shape[0])], tsem)
        for dst, s_ in zip((wr_v, wc_v, wh_v, wx_v),
                           (wr_hbm, wc_hbm, wh_hbm, wx_hbm))
    ]
    for cp in tab_cp:
        cp.start()
    for cp in tab_cp:
        cp.wait()

    def build_row(jl, carry):
        j = sid * _ROWS_PER_TILE + jl
        r = j // (_N_CHARGE * _N_HYB * _N_CHIR)
        c = (j // (_N_HYB * _N_CHIR)) % _N_CHARGE
        h = (j // _N_CHIR) % _N_HYB
        x = j % _N_CHIR
        for k in range(_D // 16):
            v = (wr_v[pl.ds(r * _D + 16 * k, 16)]
                 + wc_v[pl.ds(c * _D + 16 * k, 16)]
                 + wh_v[pl.ds(h * _D + 16 * k, 16)]
                 + wx_v[pl.ds(x * _D + 16 * k, 16)])
            tmp_v[jl, pl.ds(16 * k, 16)] = v
        return carry

    lax.fori_loop(0, _ROWS_PER_TILE, build_row, 0)
    pltpu.sync_copy(
        tmp_v, wcomb_sh.at[pl.ds(sid * _ROWS_PER_TILE, _ROWS_PER_TILE)])

    for cp in idx_cp:
        cp.wait()

    def fuse_at(row, col):
        s = pl.ds(col, 16)
        cidx_v[pl.ds(row * _L + col, 16)] = (
            (idx4_v[0, row, s] * (_N_CHARGE * _N_HYB * _N_CHIR))
            + (idx4_v[1, row, s] * (_N_HYB * _N_CHIR))
            + (idx4_v[2, row, s] * _N_CHIR)
            + idx4_v[3, row, s])

    def fuse_row(row):
        for t in range(_L // 16):
            fuse_at(row, t * 16)
        fuse_at(row, _L - 16)

    for row0 in range(4):
        fuse_row(row0)

    plsc.subcore_barrier()

    ssems = [s0, s1]
    gsems = [g0, g1]

    def scat(b, g):
        base = tile_base + g * _CHUNK
        return pltpu.make_async_copy(
            rows_v.at[b], out_hbm.at[pl.ds(base, _CHUNK)], ssems[b])

    def gath(b, g):
        return pltpu.make_async_copy(
            wcomb_sh.at[cidx_v.at[pl.ds(g * _CHUNK, _CHUNK)]],
            rows_v.at[b], gsems[b])

    def step(i, carry):
        for b in range(_NBUF):
            g = i * _NBUF + b

            @pl.when(i > 0)
            def _():
                scat(b, g - 2).wait()
                if b == 0:
                    gath(b, g).start()

            if b == 0:
                @pl.when(i == 0)
                def _():
                    gath(b, g).start()
            else:
                gath(b, g).start()

            @pl.when(4 + g < _BROWS)
            def _():
                fuse_row(4 + g)

            if b == 0:
                @pl.when(i > 0)
                def _():
                    gath(1, g - 1).wait()
                    scat(1, g - 1).start()
            else:
                gath(0, g - 1).wait()
                scat(0, g - 1).start()
        return carry

    lax.fori_loop(0, _NCHUNKS // _NBUF, step, 0)
    gath(1, _NCHUNKS - 1).wait()
    scat(1, _NCHUNKS - 1).start()
    scat(0, 0).wait()
    scat(1, 0).wait()


_sc_kernel = functools.partial(
    pl.kernel,
    out_type=jax.ShapeDtypeStruct((_BL, _D), jnp.float32),
    mesh=plsc.VectorSubcoreMesh(core_axis_name="c", subcore_axis_name="s"),
    scratch_types=[
        pltpu.VMEM((4, _BROWS, _L), jnp.int32),
        pltpu.VMEM((_PER_W,), jnp.int32),
        pltpu.VMEM((16 * _D,), jnp.float32),
        pltpu.VMEM((16 * _D,), jnp.float32),
        pltpu.VMEM((16 * _D,), jnp.float32),
        pltpu.VMEM((16 * _D,), jnp.float32),
        pltpu.VMEM((_ROWS_PER_TILE, _D), jnp.float32),
        pltpu.VMEM((_NBUF, _CHUNK, _D), jnp.float32),
        pltpu.VMEM_SHARED((_NCOMB_PAD, _D), jnp.float32),
        pltpu.SemaphoreType.DMA,
        pltpu.SemaphoreType.DMA,
        pltpu.SemaphoreType.DMA,
        pltpu.SemaphoreType.DMA,
        pltpu.SemaphoreType.DMA,
        pltpu.SemaphoreType.DMA,
    ],
)(_sc_body)


@jax.jit
def kernel(prop_atom_in_ring, prop_atom_charge, prop_atom_hybridization,
           prop_atom_chirality, W_in_ring, W_charge, W_hybridization,
           W_chirality):
    r = prop_atom_in_ring.astype(jnp.int32)
    c = prop_atom_charge.astype(jnp.int32)
    h = prop_atom_hybridization.astype(jnp.int32)
    x = prop_atom_chirality.astype(jnp.int32)
    out = _sc_kernel(r, c, h, x,
                     W_in_ring.reshape(-1), W_charge.reshape(-1),
                     W_hybridization.reshape(-1), W_chirality.reshape(-1))
    return out.reshape(_B, _L, _D)


# speedup vs baseline: 1.1045x; 1.0036x over previous
import functools

import jax
import jax.numpy as jnp
from jax import lax
from jax.experimental import pallas as pl
from jax.experimental.pallas import tpu as pltpu
from jax.experimental.pallas import tpu_sc as plsc

_B, _L, _D = 1024, 200, 128
_BL = _B * _L
_N_RING, _N_CHARGE, _N_HYB, _N_CHIR = 3, 4, 9, 5
_NCOMB_PAD = 544

_NC, _NS = 2, 16
_NW = _NC * _NS
_PER_W = _BL // _NW
_CHUNK = 128
_NCHUNKS = _PER_W // _CHUNK
_ROWS_PER_TILE = _NCOMB_PAD // _NS
_BROWS = _B // _NW
_NBUF = 2


def _sc_body(ring_hbm, charge_hbm, hyb_hbm, chir_hbm,
             wr_hbm, wc_hbm, wh_hbm, wx_hbm,
             out_hbm,
             idx4_v, cidx_v, wr_v, wc_v, wh_v, wx_v, tmp_v, rows_v, wcomb_sh,
             isem, tsem, g0, g1, s0, s1):
    cid = lax.axis_index("c")
    sid = lax.axis_index("s")
    wid = sid * _NC + cid
    tile_base = wid * _PER_W

    row_base = wid * _BROWS
    idx_cp = [
        pltpu.make_async_copy(src.at[pl.ds(row_base, _BROWS)],
                              idx4_v.at[i], isem)
        for i, src in enumerate((ring_hbm, charge_hbm, hyb_hbm, chir_hbm))
    ]
    for cp in idx_cp:
        cp.start()

    tab_cp = [
        pltpu.make_async_copy(s_, dst.at[pl.ds(0, s_.shape[0])], tsem)
        for dst, s_ in zip((wr_v, wc_v, wh_v, wx_v),
                           (wr_hbm, wc_hbm, wh_hbm, wx_hbm))
    ]
    for cp in tab_cp:
        cp.start()
    for cp in tab_cp:
        cp.wait()

    def build_row(jl, carry):
        j = sid * _ROWS_PER_TILE + jl
        r = j // (_N_CHARGE * _N_HYB * _N_CHIR)
        c = (j // (_N_HYB * _N_CHIR)) % _N_CHARGE
        h = (j // _N_CHIR) % _N_HYB
        x = j % _N_CHIR
        for k in range(_D // 16):
            v = (wr_v[pl.ds(r * _D + 16 * k, 16)]
                 + wc_v[pl.ds(c * _D + 16 * k, 16)]
                 + wh_v[pl.ds(h * _D + 16 * k, 16)]
                 + wx_v[pl.ds(x * _D + 16 * k, 16)])
            tmp_v[jl, pl.ds(16 * k, 16)] = v
        return carry

    lax.fori_loop(0, _ROWS_PER_TILE, build_row, 0)
    pltpu.sync_copy(
        tmp_v, wcomb_sh.at[pl.ds(sid * _ROWS_PER_TILE, _ROWS_PER_TILE)])

    for cp in idx_cp:
        cp.wait()

    def fuse_at(row, col):
        s = pl.ds(col, 16)
        cidx_v[pl.ds(row * _L + col, 16)] = (
            (idx4_v[0, row, s] * (_N_CHARGE * _N_HYB * _N_CHIR))
            + (idx4_v[1, row, s] * (_N_HYB * _N_CHIR))
            + (idx4_v[2, row, s] * _N_CHIR)
            + idx4_v[3, row, s])

    def fuse_row(row):
        for t in range(_L // 16):
            fuse_at(row, t * 16)
        fuse_at(row, _L - 16)

    for row0 in range(4):
        fuse_row(row0)

    plsc.subcore_barrier()

    ssems = [s0, s1]
    gsems = [g0, g1]

    def scat(b, g):
        base = tile_base + g * _CHUNK
        return pltpu.make_async_copy(
            rows_v.at[b], out_hbm.at[pl.ds(base, _CHUNK)], ssems[b])

    def gath(b, g):
        return pltpu.make_async_copy(
            wcomb_sh.at[cidx_v.at[pl.ds(g * _CHUNK, _CHUNK)]],
            rows_v.at[b], gsems[b])

    def step(i, carry):
        for b in range(_NBUF):
            g = i * _NBUF + b

            @pl.when(i > 0)
            def _():
                scat(b, g - 2).wait()
                if b == 0:
                    gath(b, g).start()

            if b == 0:
                @pl.when(i == 0)
                def _():
                    gath(b, g).start()
            else:
                gath(b, g).start()

            @pl.when(4 + g < _BROWS)
            def _():
                fuse_row(4 + g)

            if b == 0:
                @pl.when(i > 0)
                def _():
                    gath(1, g - 1).wait()
                    scat(1, g - 1).start()
            else:
                gath(0, g - 1).wait()
                scat(0, g - 1).start()
        return carry

    lax.fori_loop(0, _NCHUNKS // _NBUF, step, 0)
    gath(1, _NCHUNKS - 1).wait()
    scat(1, _NCHUNKS - 1).start()
    scat(0, 0).wait()
    scat(1, 0).wait()


_sc_kernel = functools.partial(
    pl.kernel,
    out_type=jax.ShapeDtypeStruct((_BL, _D), jnp.float32),
    mesh=plsc.VectorSubcoreMesh(core_axis_name="c", subcore_axis_name="s"),
    scratch_types=[
        pltpu.VMEM((4, _BROWS, _L), jnp.int32),
        pltpu.VMEM((_PER_W,), jnp.int32),
        pltpu.VMEM((16 * _D,), jnp.float32),
        pltpu.VMEM((16 * _D,), jnp.float32),
        pltpu.VMEM((16 * _D,), jnp.float32),
        pltpu.VMEM((16 * _D,), jnp.float32),
        pltpu.VMEM((_ROWS_PER_TILE, _D), jnp.float32),
        pltpu.VMEM((_NBUF, _CHUNK, _D), jnp.float32),
        pltpu.VMEM_SHARED((_NCOMB_PAD, _D), jnp.float32),
        pltpu.SemaphoreType.DMA,
        pltpu.SemaphoreType.DMA,
        pltpu.SemaphoreType.DMA,
        pltpu.SemaphoreType.DMA,
        pltpu.SemaphoreType.DMA,
        pltpu.SemaphoreType.DMA,
    ],
)(_sc_body)


@jax.jit
def kernel(prop_atom_in_ring, prop_atom_charge, prop_atom_hybridization,
           prop_atom_chirality, W_in_ring, W_charge, W_hybridization,
           W_chirality):
    def as_i32(a):
        return a if a.dtype == jnp.int32 else a.astype(jnp.int32)

    r = as_i32(prop_atom_in_ring)
    c = as_i32(prop_atom_charge)
    h = as_i32(prop_atom_hybridization)
    x = as_i32(prop_atom_chirality)
    out = _sc_kernel(r, c, h, x,
                     W_in_ring.reshape(-1), W_charge.reshape(-1),
                     W_hybridization.reshape(-1), W_chirality.reshape(-1))
    return out.reshape(_B, _L, _D)
